# initial kernel scaffold (unmeasured)
import jax
import jax.numpy as jnp
from jax import lax
from jax.experimental import pallas as pl
from jax.experimental.pallas import tpu as pltpu

N_DEV = 4


def kernel(A, B):
    m, k = A.shape
    _, n = B.shape
    chunk = m // N_DEV

    def body(a_hbm, b_ref, out_ref, comm_ref, a_tile, send_sems, recv_sems, load_sem):
        my = lax.axis_index("i")
        left = lax.rem(my + N_DEV - 1, N_DEV)
        right = lax.rem(my + 1, N_DEV)

        barrier_sem = pltpu.get_barrier_semaphore()
        for nbr in (left, right):
            pl.semaphore_signal(
                barrier_sem, inc=1,
                device_id=(nbr,), device_id_type=pl.DeviceIdType.MESH,
            )
        pl.semaphore_wait(barrier_sem, 2)

        def load_a_chunk(c):
            cp = pltpu.make_async_copy(
                a_hbm.at[pl.ds(c * chunk, chunk), :], a_tile, load_sem
            )
            cp.start()
            cp.wait()

        def compute_partial_into_out():
            out_ref[...] = jnp.dot(
                a_tile[...], b_ref[...], preferred_element_type=jnp.float32
            )

        load_a_chunk(lax.rem(my + N_DEV - 1, N_DEV))
        compute_partial_into_out()
        comm_ref[0, :, :] = out_ref[...]

        for h in range(N_DEV - 1):
            send_slot = h % 2
            recv_slot = (h + 1) % 2
            rdma = pltpu.make_async_remote_copy(
                src_ref=comm_ref.at[send_slot],
                dst_ref=comm_ref.at[recv_slot],
                send_sem=send_sems.at[send_slot],
                recv_sem=recv_sems.at[recv_slot],
                device_id=(right,),
                device_id_type=pl.DeviceIdType.MESH,
            )
            rdma.start()
            load_a_chunk(lax.rem(my + 2 * N_DEV - h - 2, N_DEV))
            compute_partial_into_out()
            rdma.wait()
            if h < N_DEV - 2:
                comm_ref[recv_slot, :, :] = comm_ref[recv_slot, :, :] + out_ref[...]
            else:
                out_ref[...] = out_ref[...] + comm_ref[recv_slot, :, :]

    return pl.pallas_call(
        body,
        out_shape=jax.ShapeDtypeStruct((chunk, n), jnp.float32),
        in_specs=[
            pl.BlockSpec(memory_space=pltpu.ANY),
            pl.BlockSpec(memory_space=pltpu.VMEM),
        ],
        out_specs=pl.BlockSpec(memory_space=pltpu.VMEM),
        scratch_shapes=[
            pltpu.VMEM((2, chunk, n), jnp.float32),
            pltpu.VMEM((chunk, k), jnp.float32),
            pltpu.SemaphoreType.DMA((2,)),
            pltpu.SemaphoreType.DMA((2,)),
            pltpu.SemaphoreType.DMA,
        ],
        compiler_params=pltpu.CompilerParams(collective_id=0),
    )(A, B)


# baseline (device time: 341903 ns/iter reference)
import jax
import jax.numpy as jnp
from jax import lax
from jax.experimental import pallas as pl
from jax.experimental.pallas import tpu as pltpu

N_DEV = 4


def kernel(A, B):
    m, k = A.shape
    _, n = B.shape
    chunk = m // N_DEV

    def body(a_hbm, b_ref, out_ref, comm_ref, a_tile, send_sems, recv_sems, load_sem):
        my = lax.axis_index("i")
        left = lax.rem(my + N_DEV - 1, N_DEV)
        right = lax.rem(my + 1, N_DEV)

        barrier_sem = pltpu.get_barrier_semaphore()
        for nbr in (left, right):
            pl.semaphore_signal(
                barrier_sem, inc=1,
                device_id=(nbr,), device_id_type=pl.DeviceIdType.MESH,
            )
        pl.semaphore_wait(barrier_sem, 2)

        def load_a_chunk(c):
            cp = pltpu.make_async_copy(
                a_hbm.at[pl.ds(c * chunk, chunk), :], a_tile, load_sem
            )
            cp.start()
            cp.wait()

        def compute_partial_into_out():
            out_ref[...] = jnp.dot(
                a_tile[...], b_ref[...], preferred_element_type=jnp.float32
            )

        load_a_chunk(lax.rem(my + N_DEV - 1, N_DEV))
        compute_partial_into_out()
        comm_ref[0, :, :] = out_ref[...]

        for h in range(N_DEV - 1):
            send_slot = h % 2
            recv_slot = (h + 1) % 2
            rdma = pltpu.make_async_remote_copy(
                src_ref=comm_ref.at[send_slot],
                dst_ref=comm_ref.at[recv_slot],
                send_sem=send_sems.at[send_slot],
                recv_sem=recv_sems.at[recv_slot],
                device_id=(right,),
                device_id_type=pl.DeviceIdType.MESH,
            )
            rdma.start()
            load_a_chunk(lax.rem(my + 2 * N_DEV - h - 2, N_DEV))
            compute_partial_into_out()
            rdma.wait()
            if h < N_DEV - 2:
                comm_ref[recv_slot, :, :] = comm_ref[recv_slot, :, :] + out_ref[...]
            else:
                out_ref[...] = out_ref[...] + comm_ref[recv_slot, :, :]

    return pl.pallas_call(
        body,
        out_shape=jax.ShapeDtypeStruct((chunk, n), jnp.float32),
        in_specs=[
            pl.BlockSpec(memory_space=pltpu.MemorySpace.HBM),
            pl.BlockSpec(memory_space=pltpu.VMEM),
        ],
        out_specs=pl.BlockSpec(memory_space=pltpu.VMEM),
        scratch_shapes=[
            pltpu.VMEM((2, chunk, n), jnp.float32),
            pltpu.VMEM((chunk, k), jnp.float32),
            pltpu.SemaphoreType.DMA((2,)),
            pltpu.SemaphoreType.DMA((2,)),
            pltpu.SemaphoreType.DMA,
        ],
        compiler_params=pltpu.CompilerParams(
            collective_id=0,
            vmem_limit_bytes=60 * 1024 * 1024,
        ),
    )(A, B)


# device time: 190804 ns/iter; 1.7919x vs baseline; 1.7919x over previous
import jax
import jax.numpy as jnp
from jax import lax
from jax.experimental import pallas as pl
from jax.experimental.pallas import tpu as pltpu

N_DEV = 4


def kernel(A, B):
    m, k = A.shape
    _, n = B.shape
    chunk = m // N_DEV
    n2 = n // 2

    def body(a_hbm, b_ref, out_ref, comm_r, comm_l, a_tile,
             send_r, recv_r, send_l, recv_l, load_sems):
        my = lax.axis_index("i")
        left = lax.rem(my + N_DEV - 1, N_DEV)
        right = lax.rem(my + 1, N_DEV)

        barrier_sem = pltpu.get_barrier_semaphore()
        for nbr in (left, right):
            pl.semaphore_signal(
                barrier_sem, inc=1,
                device_id=(nbr,), device_id_type=pl.DeviceIdType.MESH,
            )
        pl.semaphore_wait(barrier_sem, 2)

        def load_a(c, slot):
            cp = pltpu.make_async_copy(
                a_hbm.at[pl.ds(c * chunk, chunk), :],
                a_tile.at[slot],
                load_sems.at[slot],
            )
            cp.start()
            return cp

        def compute_halves(h):
            cr = lax.rem(my + 2 * N_DEV - h - 1, N_DEV)
            cl = lax.rem(my + h + 1, N_DEV)
            cp0 = load_a(cr, 0)
            cp1 = load_a(cl, 1)
            cp0.wait()
            out_ref[:, :n2] = jnp.dot(
                a_tile[0], b_ref[:, :n2], preferred_element_type=jnp.float32
            )
            cp1.wait()
            out_ref[:, n2:] = jnp.dot(
                a_tile[1], b_ref[:, n2:], preferred_element_type=jnp.float32
            )

        compute_halves(0)
        comm_r[0, :, :] = out_ref[:, :n2]
        comm_l[0, :, :] = out_ref[:, n2:]

        for h in range(N_DEV - 1):
            s_slot = h % 2
            r_slot = (h + 1) % 2
            rdma_r = pltpu.make_async_remote_copy(
                src_ref=comm_r.at[s_slot],
                dst_ref=comm_r.at[r_slot],
                send_sem=send_r.at[s_slot],
                recv_sem=recv_r.at[r_slot],
                device_id=(right,),
                device_id_type=pl.DeviceIdType.MESH,
            )
            rdma_l = pltpu.make_async_remote_copy(
                src_ref=comm_l.at[s_slot],
                dst_ref=comm_l.at[r_slot],
                send_sem=send_l.at[s_slot],
                recv_sem=recv_l.at[r_slot],
                device_id=(left,),
                device_id_type=pl.DeviceIdType.MESH,
            )
            rdma_r.start()
            rdma_l.start()
            compute_halves(h + 1)
            rdma_r.wait()
            rdma_l.wait()
            if h < N_DEV - 2:
                comm_r[r_slot, :, :] = comm_r[r_slot, :, :] + out_ref[:, :n2]
                comm_l[r_slot, :, :] = comm_l[r_slot, :, :] + out_ref[:, n2:]
            else:
                out_ref[:, :n2] = out_ref[:, :n2] + comm_r[r_slot, :, :]
                out_ref[:, n2:] = out_ref[:, n2:] + comm_l[r_slot, :, :]

    return pl.pallas_call(
        body,
        out_shape=jax.ShapeDtypeStruct((chunk, n), jnp.float32),
        in_specs=[
            pl.BlockSpec(memory_space=pltpu.MemorySpace.HBM),
            pl.BlockSpec(memory_space=pltpu.VMEM),
        ],
        out_specs=pl.BlockSpec(memory_space=pltpu.VMEM),
        scratch_shapes=[
            pltpu.VMEM((2, chunk, n2), jnp.float32),
            pltpu.VMEM((2, chunk, n2), jnp.float32),
            pltpu.VMEM((2, chunk, k), jnp.float32),
            pltpu.SemaphoreType.DMA((2,)),
            pltpu.SemaphoreType.DMA((2,)),
            pltpu.SemaphoreType.DMA((2,)),
            pltpu.SemaphoreType.DMA((2,)),
            pltpu.SemaphoreType.DMA((2,)),
        ],
        compiler_params=pltpu.CompilerParams(
            collective_id=0,
            vmem_limit_bytes=60 * 1024 * 1024,
        ),
    )(A, B)


# device time: 180561 ns/iter; 1.8936x vs baseline; 1.0567x over previous
import jax
import jax.numpy as jnp
from jax import lax
from jax.experimental import pallas as pl
from jax.experimental.pallas import tpu as pltpu

N_DEV = 4
T = 4


def kernel(A, B):
    m, k = A.shape
    _, n = B.shape
    chunk = m // N_DEV
    n2 = n // 2
    ts = n2 // T

    def body(a_hbm, b_ref, out_ref, comm_r, comm_l, a_tile,
             send_r, recv_r, send_l, recv_l, load_sems):
        my = lax.axis_index("i")
        left = lax.rem(my + N_DEV - 1, N_DEV)
        right = lax.rem(my + 1, N_DEV)

        barrier_sem = pltpu.get_barrier_semaphore()
        for nbr in (left, right):
            pl.semaphore_signal(
                barrier_sem, inc=1,
                device_id=(nbr,), device_id_type=pl.DeviceIdType.MESH,
            )
        pl.semaphore_wait(barrier_sem, 2)

        def load_a(phase):
            cr = lax.rem(my + 2 * N_DEV - phase - 1, N_DEV)
            cl = lax.rem(my + phase + 1, N_DEV)
            cps = []
            for slot, c in ((0, cr), (1, cl)):
                cp = pltpu.make_async_copy(
                    a_hbm.at[pl.ds(c * chunk, chunk), :],
                    a_tile.at[slot],
                    load_sems.at[slot],
                )
                cp.start()
                cps.append(cp)
            return cps

        def rdma(ring_comm, ssems, rsems, h, t, target):
            return pltpu.make_async_remote_copy(
                src_ref=ring_comm.at[h % 2, t],
                dst_ref=ring_comm.at[(h + 1) % 2, t],
                send_sem=ssems.at[h % 2, t],
                recv_sem=rsems.at[(h + 1) % 2, t],
                device_id=(target,),
                device_id_type=pl.DeviceIdType.MESH,
            )

        def b_sub(ring, t):
            return b_ref[:, pl.ds(ring * n2 + t * ts, ts)]

        def out_sub(ring, t):
            return out_ref.at[:, pl.ds(ring * n2 + t * ts, ts)]

        cps = load_a(0)
        cps[0].wait()
        cps[1].wait()
        for t in range(T):
            comm_r[0, t] = jnp.dot(
                a_tile[0], b_sub(0, t), preferred_element_type=jnp.float32
            )
            rdma(comm_r, send_r, recv_r, 0, t, right).start()
            comm_l[0, t] = jnp.dot(
                a_tile[1], b_sub(1, t), preferred_element_type=jnp.float32
            )
            rdma(comm_l, send_l, recv_l, 0, t, left).start()

        for h in range(N_DEV - 1):
            r_slot = (h + 1) % 2
            cps = load_a(h + 1)
            cps[0].wait()
            out_ref[:, :n2] = jnp.dot(
                a_tile[0], b_ref[:, :n2], preferred_element_type=jnp.float32
            )
            cps[1].wait()
            out_ref[:, n2:] = jnp.dot(
                a_tile[1], b_ref[:, n2:], preferred_element_type=jnp.float32
            )
            for t in range(T):
                for ring, comm, ssems, rsems, tgt in (
                    (0, comm_r, send_r, recv_r, right),
                    (1, comm_l, send_l, recv_l, left),
                ):
                    rdma(comm, ssems, rsems, h, t, tgt).wait()
                    o = out_sub(ring, t)
                    if h < N_DEV - 2:
                        comm[r_slot, t] = comm[r_slot, t] + o[...]
                        rdma(comm, ssems, rsems, h + 1, t, tgt).start()
                    else:
                        o[...] = o[...] + comm[r_slot, t]

    return pl.pallas_call(
        body,
        out_shape=jax.ShapeDtypeStruct((chunk, n), jnp.float32),
        in_specs=[
            pl.BlockSpec(memory_space=pltpu.MemorySpace.HBM),
            pl.BlockSpec(memory_space=pltpu.VMEM),
        ],
        out_specs=pl.BlockSpec(memory_space=pltpu.VMEM),
        scratch_shapes=[
            pltpu.VMEM((2, T, chunk, ts), jnp.float32),
            pltpu.VMEM((2, T, chunk, ts), jnp.float32),
            pltpu.VMEM((2, chunk, k), jnp.float32),
            pltpu.SemaphoreType.DMA((2, T)),
            pltpu.SemaphoreType.DMA((2, T)),
            pltpu.SemaphoreType.DMA((2, T)),
            pltpu.SemaphoreType.DMA((2, T)),
            pltpu.SemaphoreType.DMA((2,)),
        ],
        compiler_params=pltpu.CompilerParams(
            collective_id=0,
            vmem_limit_bytes=60 * 1024 * 1024,
        ),
    )(A, B)


# device time: 109301 ns/iter; 3.1281x vs baseline; 1.6520x over previous
import jax
import jax.numpy as jnp
from jax import lax
from jax.experimental import pallas as pl
from jax.experimental.pallas import tpu as pltpu

N_DEV = 4
T = 4


def kernel(A, B):
    m, k = A.shape
    _, n = B.shape
    chunk = m // N_DEV
    n2 = n // 2
    ts = n2 // T

    def body(a_hbm, b_hbm, out_ref, comm_r, comm_l,
             a_stage, a16, b_stage, b16,
             send_r, recv_r, send_l, recv_l, load_sems, b_sem):
        my = lax.axis_index("i")
        left = lax.rem(my + N_DEV - 1, N_DEV)
        right = lax.rem(my + 1, N_DEV)

        barrier_sem = pltpu.get_barrier_semaphore()
        for nbr in (left, right):
            pl.semaphore_signal(
                barrier_sem, inc=1,
                device_id=(nbr,), device_id_type=pl.DeviceIdType.MESH,
            )
        pl.semaphore_wait(barrier_sem, 2)

        def load_a(phase):
            cr = lax.rem(my + 2 * N_DEV - phase - 1, N_DEV)
            cl = lax.rem(my + phase + 1, N_DEV)
            cps = []
            for slot, c in ((0, cr), (1, cl)):
                cp = pltpu.make_async_copy(
                    a_hbm.at[pl.ds(c * chunk, chunk), :],
                    a_stage.at[slot],
                    load_sems.at[slot],
                )
                cp.start()
                cps.append(cp)
            return cps

        def cast_a():
            a16[0] = a_stage[0].astype(jnp.bfloat16)
            a16[1] = a_stage[1].astype(jnp.bfloat16)

        def cast_b_tile(ring, t):
            c0 = ring * n2 + t * ts
            cp = pltpu.make_async_copy(
                b_hbm.at[:, pl.ds(c0, ts)], b_stage, b_sem
            )
            cp.start()
            cp.wait()
            b16[:, pl.ds(c0, ts)] = b_stage[...].astype(jnp.bfloat16)

        def rdma(ring_comm, ssems, rsems, h, t, target):
            return pltpu.make_async_remote_copy(
                src_ref=ring_comm.at[h % 2, t],
                dst_ref=ring_comm.at[(h + 1) % 2, t],
                send_sem=ssems.at[h % 2, t],
                recv_sem=rsems.at[(h + 1) % 2, t],
                device_id=(target,),
                device_id_type=pl.DeviceIdType.MESH,
            )

        def b16_sub(ring, t):
            return b16[:, pl.ds(ring * n2 + t * ts, ts)]

        def out_sub(ring, t):
            return out_ref.at[:, pl.ds(ring * n2 + t * ts, ts)]

        cps = load_a(0)
        cps[0].wait()
        cps[1].wait()
        cast_a()
        for t in range(T):
            cast_b_tile(0, t)
            comm_r[0, t] = jnp.dot(
                a16[0], b16_sub(0, t), preferred_element_type=jnp.float32
            ).astype(jnp.bfloat16)
            rdma(comm_r, send_r, recv_r, 0, t, right).start()
            cast_b_tile(1, t)
            comm_l[0, t] = jnp.dot(
                a16[1], b16_sub(1, t), preferred_element_type=jnp.float32
            ).astype(jnp.bfloat16)
            rdma(comm_l, send_l, recv_l, 0, t, left).start()

        for h in range(N_DEV - 1):
            r_slot = (h + 1) % 2
            cps = load_a(h + 1)
            cps[0].wait()
            cps[1].wait()
            cast_a()
            out_ref[:, :n2] = jnp.dot(
                a16[0], b16[:, :n2], preferred_element_type=jnp.float32
            )
            out_ref[:, n2:] = jnp.dot(
                a16[1], b16[:, n2:], preferred_element_type=jnp.float32
            )
            for t in range(T):
                for ring, comm, ssems, rsems, tgt in (
                    (0, comm_r, send_r, recv_r, right),
                    (1, comm_l, send_l, recv_l, left),
                ):
                    rdma(comm, ssems, rsems, h, t, tgt).wait()
                    o = out_sub(ring, t)
                    if h < N_DEV - 2:
                        comm[r_slot, t] = (
                            comm[r_slot, t].astype(jnp.float32) + o[...]
                        ).astype(jnp.bfloat16)
                        rdma(comm, ssems, rsems, h + 1, t, tgt).start()
                    else:
                        o[...] = o[...] + comm[r_slot, t].astype(jnp.float32)

    return pl.pallas_call(
        body,
        out_shape=jax.ShapeDtypeStruct((chunk, n), jnp.float32),
        in_specs=[
            pl.BlockSpec(memory_space=pltpu.MemorySpace.HBM),
            pl.BlockSpec(memory_space=pltpu.MemorySpace.HBM),
        ],
        out_specs=pl.BlockSpec(memory_space=pltpu.VMEM),
        scratch_shapes=[
            pltpu.VMEM((2, T, chunk, ts), jnp.bfloat16),
            pltpu.VMEM((2, T, chunk, ts), jnp.bfloat16),
            pltpu.VMEM((2, chunk, k), jnp.float32),
            pltpu.VMEM((2, chunk, k), jnp.bfloat16),
            pltpu.VMEM((k, ts), jnp.float32),
            pltpu.VMEM((k, n), jnp.bfloat16),
            pltpu.SemaphoreType.DMA((2, T)),
            pltpu.SemaphoreType.DMA((2, T)),
            pltpu.SemaphoreType.DMA((2, T)),
            pltpu.SemaphoreType.DMA((2, T)),
            pltpu.SemaphoreType.DMA((2,)),
            pltpu.SemaphoreType.DMA,
        ],
        compiler_params=pltpu.CompilerParams(
            collective_id=0,
            vmem_limit_bytes=60 * 1024 * 1024,
        ),
    )(A, B)


# device time: 107509 ns/iter; 3.1802x vs baseline; 1.0167x over previous
import jax
import jax.numpy as jnp
from jax import lax
from jax.experimental import pallas as pl
from jax.experimental.pallas import tpu as pltpu

N_DEV = 4
T = 4


def kernel(A, B):
    m, k = A.shape
    _, n = B.shape
    chunk = m // N_DEV
    n2 = n // 2
    ts = n2 // T

    def body(a_hbm, b_hbm, out_ref, comm_r, comm_l,
             a_stage, a16, b_stage, b16,
             send_r, recv_r, send_l, recv_l, load_sems, b_sem):
        my = lax.axis_index("i")
        left = lax.rem(my + N_DEV - 1, N_DEV)
        right = lax.rem(my + 1, N_DEV)

        barrier_sem = pltpu.get_barrier_semaphore()
        for nbr in (left, right):
            pl.semaphore_signal(
                barrier_sem, inc=1,
                device_id=(nbr,), device_id_type=pl.DeviceIdType.MESH,
            )
        pl.semaphore_wait(barrier_sem, 2)

        def chunk_r(phase):
            return lax.rem(my + 2 * N_DEV - phase - 1, N_DEV)

        def chunk_l(phase):
            return lax.rem(my + phase + 1, N_DEV)

        def load_a(c, slot):
            cp = pltpu.make_async_copy(
                a_hbm.at[pl.ds(c * chunk, chunk), :],
                a_stage.at[slot],
                load_sems.at[slot],
            )
            cp.start()
            return cp

        def cast_a(c, slot):
            a16[c] = a_stage[slot].astype(jnp.bfloat16)

        def cast_b_tile(ring, t):
            c0 = ring * n2 + t * ts
            cp = pltpu.make_async_copy(
                b_hbm.at[:, pl.ds(c0, ts)], b_stage, b_sem
            )
            cp.start()
            cp.wait()
            b16[:, pl.ds(c0, ts)] = b_stage[...].astype(jnp.bfloat16)

        def rdma(ring_comm, ssems, rsems, h, t, target):
            return pltpu.make_async_remote_copy(
                src_ref=ring_comm.at[h % 2, t],
                dst_ref=ring_comm.at[(h + 1) % 2, t],
                send_sem=ssems.at[h % 2, t],
                recv_sem=rsems.at[(h + 1) % 2, t],
                device_id=(target,),
                device_id_type=pl.DeviceIdType.MESH,
            )

        def b16_sub(ring, t):
            return b16[:, pl.ds(ring * n2 + t * ts, ts)]

        def out_sub(ring, t):
            return out_ref.at[:, pl.ds(ring * n2 + t * ts, ts)]

        cr0, cl0 = chunk_r(0), chunk_l(0)
        cp0 = load_a(cr0, 0)
        cp1 = load_a(cl0, 1)
        cp0.wait()
        cast_a(cr0, 0)
        cp1.wait()
        cast_a(cl0, 1)
        for t in range(T):
            cast_b_tile(0, t)
            comm_r[0, t] = jnp.dot(
                a16[cr0], b16_sub(0, t), preferred_element_type=jnp.float32
            ).astype(jnp.bfloat16)
            rdma(comm_r, send_r, recv_r, 0, t, right).start()
            cast_b_tile(1, t)
            comm_l[0, t] = jnp.dot(
                a16[cl0], b16_sub(1, t), preferred_element_type=jnp.float32
            ).astype(jnp.bfloat16)
            rdma(comm_l, send_l, recv_l, 0, t, left).start()
            if t == 0:
                cp0 = load_a(chunk_r(1), 0)
                cp1 = load_a(chunk_r(3), 1)
        cp0.wait()
        cast_a(chunk_r(1), 0)
        cp1.wait()
        cast_a(chunk_r(3), 1)

        for h in range(N_DEV - 1):
            r_slot = (h + 1) % 2
            out_ref[:, :n2] = jnp.dot(
                a16[chunk_r(h + 1)], b16[:, :n2],
                preferred_element_type=jnp.float32,
            )
            out_ref[:, n2:] = jnp.dot(
                a16[chunk_l(h + 1)], b16[:, n2:],
                preferred_element_type=jnp.float32,
            )
            for t in range(T):
                for ring, comm, ssems, rsems, tgt in (
                    (0, comm_r, send_r, recv_r, right),
                    (1, comm_l, send_l, recv_l, left),
                ):
                    rdma(comm, ssems, rsems, h, t, tgt).wait()
                    o = out_sub(ring, t)
                    if h < N_DEV - 2:
                        comm[r_slot, t] = (
                            comm[r_slot, t].astype(jnp.float32) + o[...]
                        ).astype(jnp.bfloat16)
                        rdma(comm, ssems, rsems, h + 1, t, tgt).start()
                    else:
                        o[...] = o[...] + comm[r_slot, t].astype(jnp.float32)

    return pl.pallas_call(
        body,
        out_shape=jax.ShapeDtypeStruct((chunk, n), jnp.float32),
        in_specs=[
            pl.BlockSpec(memory_space=pltpu.MemorySpace.HBM),
            pl.BlockSpec(memory_space=pltpu.MemorySpace.HBM),
        ],
        out_specs=pl.BlockSpec(memory_space=pltpu.VMEM),
        scratch_shapes=[
            pltpu.VMEM((2, T, chunk, ts), jnp.bfloat16),
            pltpu.VMEM((2, T, chunk, ts), jnp.bfloat16),
            pltpu.VMEM((2, chunk, k), jnp.float32),
            pltpu.VMEM((N_DEV, chunk, k), jnp.bfloat16),
            pltpu.VMEM((k, ts), jnp.float32),
            pltpu.VMEM((k, n), jnp.bfloat16),
            pltpu.SemaphoreType.DMA((2, T)),
            pltpu.SemaphoreType.DMA((2, T)),
            pltpu.SemaphoreType.DMA((2, T)),
            pltpu.SemaphoreType.DMA((2, T)),
            pltpu.SemaphoreType.DMA((2,)),
            pltpu.SemaphoreType.DMA,
        ],
        compiler_params=pltpu.CompilerParams(
            collective_id=0,
            vmem_limit_bytes=60 * 1024 * 1024,
        ),
    )(A, B)


# device time: 99253 ns/iter; 3.4448x vs baseline; 1.0832x over previous
import jax
import jax.numpy as jnp
from jax import lax
from jax.experimental import pallas as pl
from jax.experimental.pallas import tpu as pltpu

N_DEV = 4
T = 4


def kernel(A, B):
    m, k = A.shape
    _, n = B.shape
    chunk = m // N_DEV
    n2 = n // 2
    ts = n2 // T

    def body(a_hbm, b_hbm, out_ref, comm_r, comm_l,
             a_stage, a16, b_stage, b16,
             send_r, recv_r, send_l, recv_l, load_sems, b_sems):
        my = lax.axis_index("i")
        left = lax.rem(my + N_DEV - 1, N_DEV)
        right = lax.rem(my + 1, N_DEV)

        barrier_sem = pltpu.get_barrier_semaphore()
        for nbr in (left, right):
            pl.semaphore_signal(
                barrier_sem, inc=1,
                device_id=(nbr,), device_id_type=pl.DeviceIdType.MESH,
            )
        pl.semaphore_wait(barrier_sem, 2)

        def chunk_r(phase):
            return lax.rem(my + 2 * N_DEV - phase - 1, N_DEV)

        def chunk_l(phase):
            return lax.rem(my + phase + 1, N_DEV)

        def load_a(c, slot):
            cp = pltpu.make_async_copy(
                a_hbm.at[pl.ds(c * chunk, chunk), :],
                a_stage.at[slot],
                load_sems.at[slot],
            )
            cp.start()
            return cp

        def cast_a(c, slot):
            a16[c] = a_stage[slot].astype(jnp.bfloat16)

        def b_dma(ring, t, slot):
            return pltpu.make_async_copy(
                b_hbm.at[:, pl.ds(ring * n2 + t * ts, ts)],
                b_stage.at[slot],
                b_sems.at[slot],
            )

        def rdma(ring, h, t):
            comm = (comm_r, comm_l)[ring]
            ssems = (send_r, send_l)[ring]
            rsems = (recv_r, recv_l)[ring]
            return pltpu.make_async_remote_copy(
                src_ref=comm.at[h % 2, t],
                dst_ref=comm.at[(h + 1) % 2, t],
                send_sem=ssems.at[h % 2, t],
                recv_sem=rsems.at[(h + 1) % 2, t],
                device_id=((right, left)[ring],),
                device_id_type=pl.DeviceIdType.MESH,
            )

        def b16_sub(ring, t):
            return b16[:, pl.ds(ring * n2 + t * ts, ts)]

        def out_sub(ring, t):
            return out_ref.at[:, pl.ds(ring * n2 + t * ts, ts)]

        pairs = [(ring, t) for t in range(T) for ring in (0, 1)]

        cr0, cl0 = chunk_r(0), chunk_l(0)
        cpa0 = load_a(cr0, 0)
        cpa1 = load_a(cl0, 1)
        cpb = b_dma(0, 0, 0)
        cpb.start()
        cpa0.wait()
        cast_a(cr0, 0)
        cpa1.wait()
        cast_a(cl0, 1)
        bslot = 0
        for i, (ring, t) in enumerate(pairs):
            cpb.wait()
            cur = bslot
            if i + 1 < len(pairs):
                nring, nt = pairs[i + 1]
                bslot = 1 - bslot
                cpb = b_dma(nring, nt, bslot)
                cpb.start()
            b16[:, pl.ds(ring * n2 + t * ts, ts)] = (
                b_stage[cur].astype(jnp.bfloat16)
            )
            a_idx = cr0 if ring == 0 else cl0
            comm = (comm_r, comm_l)[ring]
            comm[0, t] = jnp.dot(
                a16[a_idx], b16_sub(ring, t),
                preferred_element_type=jnp.float32,
            ).astype(jnp.bfloat16)
            rdma(ring, 0, t).start()
            if i == 1:
                cpa0 = load_a(chunk_r(1), 0)
                cpa1 = load_a(chunk_r(3), 1)
        cpa0.wait()
        cast_a(chunk_r(1), 0)
        cpa1.wait()
        cast_a(chunk_r(3), 1)

        for h in range(N_DEV - 1):
            r_slot = (h + 1) % 2
            ca = chunk_r(h + 1)
            cl_ = chunk_l(h + 1)
            for ring, t in pairs:
                a_idx = ca if ring == 0 else cl_
                o = out_sub(ring, t)
                o[...] = jnp.dot(
                    a16[a_idx], b16_sub(ring, t),
                    preferred_element_type=jnp.float32,
                )
                rdma(ring, h, t).wait()
                comm = (comm_r, comm_l)[ring]
                if h < N_DEV - 2:
                    comm[r_slot, t] = (
                        comm[r_slot, t].astype(jnp.float32) + o[...]
                    ).astype(jnp.bfloat16)
                    rdma(ring, h + 1, t).start()
                else:
                    o[...] = o[...] + comm[r_slot, t].astype(jnp.float32)

    return pl.pallas_call(
        body,
        out_shape=jax.ShapeDtypeStruct((chunk, n), jnp.float32),
        in_specs=[
            pl.BlockSpec(memory_space=pltpu.MemorySpace.HBM),
            pl.BlockSpec(memory_space=pltpu.MemorySpace.HBM),
        ],
        out_specs=pl.BlockSpec(memory_space=pltpu.VMEM),
        scratch_shapes=[
            pltpu.VMEM((2, T, chunk, ts), jnp.bfloat16),
            pltpu.VMEM((2, T, chunk, ts), jnp.bfloat16),
            pltpu.VMEM((2, chunk, k), jnp.float32),
            pltpu.VMEM((N_DEV, chunk, k), jnp.bfloat16),
            pltpu.VMEM((2, k, ts), jnp.float32),
            pltpu.VMEM((k, n), jnp.bfloat16),
            pltpu.SemaphoreType.DMA((2, T)),
            pltpu.SemaphoreType.DMA((2, T)),
            pltpu.SemaphoreType.DMA((2, T)),
            pltpu.SemaphoreType.DMA((2, T)),
            pltpu.SemaphoreType.DMA((2,)),
            pltpu.SemaphoreType.DMA((2,)),
        ],
        compiler_params=pltpu.CompilerParams(
            collective_id=0,
            vmem_limit_bytes=60 * 1024 * 1024,
        ),
    )(A, B)
